# Initial kernel scaffold; baseline (speedup 1.0000x reference)
import jax
import jax.numpy as jnp
from jax.experimental import pallas as pl


def kernel(dense, sparse, tables, W1, b1, W2, b2, W3, b3):
    raise NotImplementedError("write your pallas kernel here")


# speedup vs baseline: 2.1768x; 2.1768x over previous
import functools

import jax
import jax.numpy as jnp
from jax import lax
from jax.experimental import pallas as pl
from jax.experimental.pallas import tpu as pltpu
from jax.experimental.pallas import tpu_sc as plsc

B = 4096
DENSE_DIM = 13
N_FIELDS = 26
VOCAB = 100000
EMBED_DIM = 32
BF = B * N_FIELDS

_LANES = 16
_GCHUNK = 128


def _sc_gather_make(num_workers: int, per_w: int):
  mesh = plsc.VectorSubcoreMesh(core_axis_name="c", subcore_axis_name="s")

  @functools.partial(
      pl.kernel,
      mesh=mesh,
      compiler_params=pltpu.CompilerParams(use_tc_tiling_on_sc=False),
      out_type=jax.ShapeDtypeStruct((BF, EMBED_DIM), jnp.float32),
      scratch_types=[
          pltpu.VMEM((per_w,), jnp.int32),
          pltpu.VMEM((per_w, EMBED_DIM), jnp.float32),
          pltpu.SemaphoreType.DMA,
      ],
  )
  def gather_k(tab_hbm, sp_hbm, out_hbm, idx_v, rows_v, sem):
    wid = lax.axis_index("s") * 2 + lax.axis_index("c")
    base = wid * per_w
    pltpu.sync_copy(sp_hbm.at[pl.ds(base, per_w)], idx_v)
    iota = lax.iota(jnp.int32, _LANES)
    def idx_body(k, carry):
      sl = pl.ds(k * _LANES, _LANES)
      pos = base + k * _LANES + iota
      idx_v[sl] = idx_v[sl] + lax.rem(pos, N_FIELDS) * VOCAB
      return carry
    lax.fori_loop(0, per_w // _LANES, idx_body, 0)
    def g_body(j, carry):
      sl = pl.ds(j * _GCHUNK, _GCHUNK)
      pltpu.async_copy(tab_hbm.at[idx_v.at[sl]], rows_v.at[sl], sem).wait()
      return carry
    lax.fori_loop(0, per_w // _GCHUNK, g_body, 0)
    pltpu.sync_copy(rows_v, out_hbm.at[pl.ds(base, per_w)])

  return gather_k


def _mlp_body(dense_ref, embs_ref, w1d_ref, w1e_ref, b1_ref, w2_ref, b2_ref,
              w3_ref, b3_ref, out_ref):
  x1 = (dense_ref[...] @ w1d_ref[...] + embs_ref[...] @ w1e_ref[...]
        + b1_ref[...])
  h1 = jnp.maximum(x1, 0.0)
  h2 = jnp.maximum(h1 @ w2_ref[...] + b2_ref[...], 0.0)
  o = h2 @ w3_ref[...] + b3_ref[...]
  out_ref[...] = jax.nn.sigmoid(o)


def kernel(dense, sparse, tables, W1, b1, W2, b2, W3, b3):
  tab_flat = tables.reshape(N_FIELDS * VOCAB, EMBED_DIM)
  sp_flat = sparse.reshape(BF)

  info = plsc.get_sparse_core_info()
  nw = info.num_cores * info.num_subcores
  per_w = BF // nw
  embs = _sc_gather_make(nw, per_w)(tab_flat, sp_flat)
  embs = embs.reshape(B, N_FIELDS * EMBED_DIM)

  w1d = W1[:DENSE_DIM]
  w1e = W1[DENSE_DIM:]
  bs = 512
  grid = (B // bs,)
  full = lambda shape: pl.BlockSpec(shape, lambda i: (0, 0))
  out = pl.pallas_call(
      _mlp_body,
      grid=grid,
      in_specs=[
          pl.BlockSpec((bs, DENSE_DIM), lambda i: (i, 0)),
          pl.BlockSpec((bs, N_FIELDS * EMBED_DIM), lambda i: (i, 0)),
          full(w1d.shape),
          full(w1e.shape),
          pl.BlockSpec((1, 128), lambda i: (0, 0)),
          full(W2.shape),
          pl.BlockSpec((1, 64), lambda i: (0, 0)),
          full(W3.shape),
          pl.BlockSpec((1, 1), lambda i: (0, 0)),
      ],
      out_specs=pl.BlockSpec((bs, 1), lambda i: (i, 0)),
      out_shape=jax.ShapeDtypeStruct((B, 1), jnp.float32),
  )(dense, embs, w1d, w1e, b1.reshape(1, 128), W2, b2.reshape(1, 64), W3,
    b3.reshape(1, 1))
  return out.reshape(B)


# speedup vs baseline: 2.2041x; 1.0125x over previous
import functools

import jax
import jax.numpy as jnp
from jax import lax
from jax.experimental import pallas as pl
from jax.experimental.pallas import tpu as pltpu
from jax.experimental.pallas import tpu_sc as plsc

B = 4096
DENSE_DIM = 13
N_FIELDS = 26
VOCAB = 100000
EMBED_DIM = 32
BF = B * N_FIELDS

_LANES = 16
_GCHUNK = 128


def _sc_gather_make(num_workers: int, per_w: int):
  mesh = plsc.VectorSubcoreMesh(core_axis_name="c", subcore_axis_name="s")

  @functools.partial(
      pl.kernel,
      mesh=mesh,
      compiler_params=pltpu.CompilerParams(use_tc_tiling_on_sc=False),
      out_type=jax.ShapeDtypeStruct((BF, EMBED_DIM), jnp.float32),
      scratch_types=[
          pltpu.VMEM((per_w,), jnp.int32),
          pltpu.VMEM((per_w, EMBED_DIM), jnp.float32),
          pltpu.SemaphoreType.DMA,
      ],
  )
  def gather_k(tab_hbm, sp_hbm, out_hbm, idx_v, rows_v, sem):
    wid = lax.axis_index("s") * 2 + lax.axis_index("c")
    base = wid * per_w
    pltpu.sync_copy(sp_hbm.at[pl.ds(base, per_w)], idx_v)
    iota = lax.iota(jnp.int32, _LANES)
    def idx_body(k, carry):
      sl = pl.ds(k * _LANES, _LANES)
      pos = base + k * _LANES + iota
      idx_v[sl] = idx_v[sl] + lax.rem(pos, N_FIELDS) * VOCAB
      return carry
    lax.fori_loop(0, per_w // _LANES, idx_body, 0)
    def g_fire(j, carry):
      sl = pl.ds(j * _GCHUNK, _GCHUNK)
      pltpu.async_copy(tab_hbm.at[idx_v.at[sl]], rows_v.at[sl], sem)
      return carry
    lax.fori_loop(0, per_w // _GCHUNK, g_fire, 0)
    def g_drain(j, carry):
      sl = pl.ds(j * _GCHUNK, _GCHUNK)
      pltpu.make_async_copy(tab_hbm.at[idx_v.at[sl]], rows_v.at[sl], sem).wait()
      return carry
    lax.fori_loop(0, per_w // _GCHUNK, g_drain, 0)
    pltpu.sync_copy(rows_v, out_hbm.at[pl.ds(base, per_w)])

  return gather_k


def _mlp_body(dense_ref, embs_ref, w1d_ref, w1e_ref, b1_ref, w2_ref, b2_ref,
              w3_ref, b3_ref, out_ref):
  x1 = (dense_ref[...] @ w1d_ref[...] + embs_ref[...] @ w1e_ref[...]
        + b1_ref[...])
  h1 = jnp.maximum(x1, 0.0)
  h2 = jnp.maximum(h1 @ w2_ref[...] + b2_ref[...], 0.0)
  o = h2 @ w3_ref[...] + b3_ref[...]
  out_ref[...] = jax.nn.sigmoid(o)


def kernel(dense, sparse, tables, W1, b1, W2, b2, W3, b3):
  tab_flat = tables.reshape(N_FIELDS * VOCAB, EMBED_DIM)
  sp_flat = sparse.reshape(BF)

  info = plsc.get_sparse_core_info()
  nw = info.num_cores * info.num_subcores
  per_w = BF // nw
  embs = _sc_gather_make(nw, per_w)(tab_flat, sp_flat)
  embs = embs.reshape(B, N_FIELDS * EMBED_DIM)

  w1d = W1[:DENSE_DIM]
  w1e = W1[DENSE_DIM:]
  bs = 512
  grid = (B // bs,)
  full = lambda shape: pl.BlockSpec(shape, lambda i: (0, 0))
  out = pl.pallas_call(
      _mlp_body,
      grid=grid,
      in_specs=[
          pl.BlockSpec((bs, DENSE_DIM), lambda i: (i, 0)),
          pl.BlockSpec((bs, N_FIELDS * EMBED_DIM), lambda i: (i, 0)),
          full(w1d.shape),
          full(w1e.shape),
          pl.BlockSpec((1, 128), lambda i: (0, 0)),
          full(W2.shape),
          pl.BlockSpec((1, 64), lambda i: (0, 0)),
          full(W3.shape),
          pl.BlockSpec((1, 1), lambda i: (0, 0)),
      ],
      out_specs=pl.BlockSpec((bs, 1), lambda i: (i, 0)),
      out_shape=jax.ShapeDtypeStruct((B, 1), jnp.float32),
  )(dense, embs, w1d, w1e, b1.reshape(1, 128), W2, b2.reshape(1, 64), W3,
    b3.reshape(1, 1))
  return out.reshape(B)
